# baseline (speedup 1.0000x reference)
import jax
import jax.numpy as jnp
from jax.experimental import pallas as pl

K_NB = 16
MT = 128
NCH = 64
LCH = 128


def _knn_block(q_ref, s_ref, qq_ref, ss_ref, vals_ref, idx_ref):
    q = q_ref[0]
    s = s_ref[0]
    n = s.shape[0]

    cross = jax.lax.dot_general(
        q, s, (((1,), (1,)), ((), ())), preferred_element_type=jnp.float32)
    qq = qq_ref[0]
    ss = ss_ref[0]
    d2 = (qq + ss) - 2.0 * cross
    dist = jnp.sqrt(jnp.maximum(d2, 0.0))

    d3 = dist.reshape(MT, NCH, LCH)
    cm = jnp.min(d3, axis=2)

    iota_c = jax.lax.broadcasted_iota(jnp.int32, (MT, NCH), 1)
    iota_c3 = jax.lax.broadcasted_iota(jnp.int32, (MT, NCH, 1), 1)
    lane = jax.lax.broadcasted_iota(jnp.int32, (MT, LCH), 1)
    inf = jnp.inf

    val_cols = []
    idx_cols = []
    prev = []
    for _ in range(K_NB):
        m = jnp.min(cm, axis=1, keepdims=True)
        amc = jnp.min(jnp.where(cm <= m, iota_c, NCH), axis=1,
                      keepdims=True)
        ext = jnp.min(jnp.where(iota_c3 == amc[:, :, None], d3, inf),
                      axis=1)
        for pc, plm in prev:
            ext = jnp.where((pc == amc) & (lane == plm), inf, ext)
        aml = jnp.min(jnp.where(ext <= m, lane, LCH), axis=1,
                      keepdims=True)
        val_cols.append(m)
        idx_cols.append(amc * LCH + aml)
        prev.append((amc, aml))
        nm = jnp.min(jnp.where(lane == aml, inf, ext), axis=1,
                     keepdims=True)
        cm = jnp.where(iota_c == amc, nm, cm)

    vals_ref[0] = jnp.concatenate(val_cols, axis=1)
    idx_ref[0] = jnp.concatenate(idx_cols, axis=1)


def kernel(query, support):
    b, m, c = query.shape
    _, n, _ = support.shape
    qq = jnp.sum(query * query, axis=-1, keepdims=True)
    ss = jnp.sum(support * support, axis=-1)[:, None, :]
    grid = (b, m // MT)
    vals, idx = pl.pallas_call(
        _knn_block,
        grid=grid,
        in_specs=[
            pl.BlockSpec((1, MT, c), lambda bi, mi: (bi, mi, 0)),
            pl.BlockSpec((1, n, c), lambda bi, mi: (bi, 0, 0)),
            pl.BlockSpec((1, MT, 1), lambda bi, mi: (bi, mi, 0)),
            pl.BlockSpec((1, 1, n), lambda bi, mi: (bi, 0, 0)),
        ],
        out_specs=[
            pl.BlockSpec((1, MT, K_NB), lambda bi, mi: (bi, mi, 0)),
            pl.BlockSpec((1, MT, K_NB), lambda bi, mi: (bi, mi, 0)),
        ],
        out_shape=[
            jax.ShapeDtypeStruct((b, m, K_NB), jnp.float32),
            jax.ShapeDtypeStruct((b, m, K_NB), jnp.int32),
        ],
    )(query, support, qq, ss)
    return (vals, idx)


# speedup vs baseline: 1.2905x; 1.2905x over previous
import jax
import jax.numpy as jnp
from jax.experimental import pallas as pl

K_NB = 16
MT = 128
LT = 128


def _knn_block(q_ref, s_ref, qq_ref, ss_ref, vals_ref, idx_ref):
    q = q_ref[0]
    s = s_ref[0]
    n = s.shape[0]
    nt = n // LT

    cross = jax.lax.dot_general(
        q, s, (((1,), (1,)), ((), ())), preferred_element_type=jnp.float32)
    qq = qq_ref[0]
    ss = ss_ref[0]
    d2 = (qq + ss) - 2.0 * cross
    dist = jnp.sqrt(jnp.maximum(d2, 0.0))

    lane_n = jax.lax.broadcasted_iota(jnp.int32, (MT, n), 1)
    lane_t = jax.lax.broadcasted_iota(jnp.int32, (MT, LT), 1)
    val_cols = []
    idx_cols = []
    for _ in range(K_NB):
        rmin = dist[:, 0:LT]
        rarg = jnp.zeros((MT, LT), jnp.int32)
        for t in range(1, nt):
            x = dist[:, t * LT:(t + 1) * LT]
            cond = x < rmin
            rmin = jnp.where(cond, x, rmin)
            rarg = jnp.where(cond, t, rarg)
        m = jnp.min(rmin, axis=1, keepdims=True)
        gidx = jnp.min(jnp.where(rmin <= m, rarg * LT + lane_t, n),
                       axis=1, keepdims=True)
        val_cols.append(m)
        idx_cols.append(gidx)
        dist = jnp.where(lane_n == gidx, jnp.inf, dist)

    vals_ref[0] = jnp.concatenate(val_cols, axis=1)
    idx_ref[0] = jnp.concatenate(idx_cols, axis=1)


def kernel(query, support):
    b, m, c = query.shape
    _, n, _ = support.shape
    qq = jnp.sum(query * query, axis=-1, keepdims=True)
    ss = jnp.sum(support * support, axis=-1)[:, None, :]
    grid = (b, m // MT)
    vals, idx = pl.pallas_call(
        _knn_block,
        grid=grid,
        in_specs=[
            pl.BlockSpec((1, MT, c), lambda bi, mi: (bi, mi, 0)),
            pl.BlockSpec((1, n, c), lambda bi, mi: (bi, 0, 0)),
            pl.BlockSpec((1, MT, 1), lambda bi, mi: (bi, mi, 0)),
            pl.BlockSpec((1, 1, n), lambda bi, mi: (bi, 0, 0)),
        ],
        out_specs=[
            pl.BlockSpec((1, MT, K_NB), lambda bi, mi: (bi, mi, 0)),
            pl.BlockSpec((1, MT, K_NB), lambda bi, mi: (bi, mi, 0)),
        ],
        out_shape=[
            jax.ShapeDtypeStruct((b, m, K_NB), jnp.float32),
            jax.ShapeDtypeStruct((b, m, K_NB), jnp.int32),
        ],
    )(query, support, qq, ss)
    return (vals, idx)


# speedup vs baseline: 1.3961x; 1.0819x over previous
import jax
import jax.numpy as jnp
from jax.experimental import pallas as pl

K_NB = 16
MT = 256
LT = 128


def _knn_block(q_ref, s_ref, qq_ref, ss_ref, vals_ref, idx_ref):
    q = q_ref[0]
    s = s_ref[0]
    n = s.shape[0]
    nt = n // LT

    cross = jax.lax.dot_general(
        q, s, (((1,), (1,)), ((), ())), preferred_element_type=jnp.float32)
    qq = qq_ref[0]
    ss = ss_ref[0]
    d2 = (qq + ss) - 2.0 * cross
    dist = jnp.sqrt(jnp.maximum(d2, 0.0))

    lane_n = jax.lax.broadcasted_iota(jnp.int32, (MT, n), 1)
    lane_t = jax.lax.broadcasted_iota(jnp.int32, (MT, LT), 1)
    val_cols = []
    idx_cols = []
    for _ in range(K_NB):
        rmin = dist[:, 0:LT]
        rarg = jnp.zeros((MT, LT), jnp.int32)
        for t in range(1, nt):
            x = dist[:, t * LT:(t + 1) * LT]
            cond = x < rmin
            rmin = jnp.where(cond, x, rmin)
            rarg = jnp.where(cond, t, rarg)
        m = jnp.min(rmin, axis=1, keepdims=True)
        gidx = jnp.min(jnp.where(rmin <= m, rarg * LT + lane_t, n),
                       axis=1, keepdims=True)
        val_cols.append(m)
        idx_cols.append(gidx)
        dist = jnp.where(lane_n == gidx, jnp.inf, dist)

    vals_ref[0] = jnp.concatenate(val_cols, axis=1)
    idx_ref[0] = jnp.concatenate(idx_cols, axis=1)


def kernel(query, support):
    b, m, c = query.shape
    _, n, _ = support.shape
    qq = jnp.sum(query * query, axis=-1, keepdims=True)
    ss = jnp.sum(support * support, axis=-1)[:, None, :]
    grid = (b, m // MT)
    vals, idx = pl.pallas_call(
        _knn_block,
        grid=grid,
        in_specs=[
            pl.BlockSpec((1, MT, c), lambda bi, mi: (bi, mi, 0)),
            pl.BlockSpec((1, n, c), lambda bi, mi: (bi, 0, 0)),
            pl.BlockSpec((1, MT, 1), lambda bi, mi: (bi, mi, 0)),
            pl.BlockSpec((1, 1, n), lambda bi, mi: (bi, 0, 0)),
        ],
        out_specs=[
            pl.BlockSpec((1, MT, K_NB), lambda bi, mi: (bi, mi, 0)),
            pl.BlockSpec((1, MT, K_NB), lambda bi, mi: (bi, mi, 0)),
        ],
        out_shape=[
            jax.ShapeDtypeStruct((b, m, K_NB), jnp.float32),
            jax.ShapeDtypeStruct((b, m, K_NB), jnp.int32),
        ],
    )(query, support, qq, ss)
    return (vals, idx)
